# baseline (speedup 1.0000x reference)
import functools

import jax
import jax.numpy as jnp
from jax import lax
from jax.experimental import pallas as pl
from jax.experimental.pallas import tpu as pltpu
from jax.experimental.pallas import tpu_sc as plsc

D_MODEL = 64
SCALE = 8.0

BATCH = 4096
SEQ = 200
NUM_CORES = 2
NUM_SUBCORES = 16
NW = NUM_CORES * NUM_SUBCORES
B_LOCAL = BATCH // NW
NBUF = 4


def _build():
  mesh = plsc.VectorSubcoreMesh(core_axis_name="c", subcore_axis_name="s")

  @functools.partial(
      pl.kernel,
      mesh=mesh,
      out_type=jax.ShapeDtypeStruct(
          (SEQ, D_MODEL // 8, BATCH // 128, 8, 128), jnp.float32
      ),
      scratch_types=[
          pltpu.VMEM((SEQ, B_LOCAL), jnp.int32),
          pltpu.VMEM((NBUF, B_LOCAL, D_MODEL), jnp.float32),
          pltpu.VMEM((NBUF, D_MODEL // 8, 8, B_LOCAL), jnp.float32),
          [pltpu.SemaphoreType.DMA] * NBUF,
          [pltpu.SemaphoreType.DMA] * NBUF,
      ],
      compiler_params=pltpu.CompilerParams(
          use_tc_tiling_on_sc=False, needs_layout_passes=False
      ),
  )
  def emb(xt_hbm, table_hbm, out_hbm, xt_v, rv, tv, gsems, ssems):
    wid = lax.axis_index("s") * NUM_CORES + lax.axis_index("c")
    b0 = wid * B_LOCAL

    pltpu.sync_copy(xt_hbm.at[:, pl.ds(b0, B_LOCAL)], xt_v)

    lanes = lax.iota(jnp.int32, 16)

    def start_gather(b, j):
      pltpu.make_async_copy(
          table_hbm.at[xt_v.at[j]], rv.at[b], gsems[b]
      ).start()

    def wait_gather(b, j):
      pltpu.make_async_copy(
          table_hbm.at[xt_v.at[j]], rv.at[b], gsems[b]
      ).wait()

    for b in range(NBUF):
      start_gather(b, b)

    def transpose_scale(b):
      def body(dt, c2):
        for di in range(8):
          dcol = jnp.full((16,), dt * 8 + di, jnp.int32)
          for g in range(B_LOCAL // 16):
            vec = plsc.load_gather(rv.at[b], [g * 16 + lanes, dcol])
            tv[b, dt, di, pl.ds(g * 16, 16)] = vec * SCALE
        return c2

      lax.fori_loop(0, D_MODEL // 8, body, 0)

    def outer(i, carry):
      for b in range(NBUF):
        j = i * NBUF + b
        wait_gather(b, j)
        transpose_scale(b)
        pltpu.make_async_copy(
            tv.at[b], out_hbm.at[j, :, wid], ssems[b]
        ).start()

        @pl.when(j + NBUF < SEQ)
        def _():
          pltpu.make_async_copy(
              tv.at[b], out_hbm.at[j, :, wid], ssems[b]
          ).wait()
          start_gather(b, j + NBUF)

      return carry

    lax.fori_loop(0, SEQ // NBUF, outer, 0)

    for b in range(NBUF):
      pltpu.make_async_copy(
          tv.at[b], out_hbm.at[0, :, wid], ssems[b]
      ).wait()

  return emb


_emb = _build()


@jax.jit
def kernel(x, lut):
  out5 = _emb(x.T, lut)
  return out5.transpose(2, 4, 0, 1, 3).reshape(BATCH, SEQ, D_MODEL)


# speedup vs baseline: 1.6219x; 1.6219x over previous
import functools

import jax
import jax.numpy as jnp
from jax import lax
from jax.experimental import pallas as pl
from jax.experimental.pallas import tpu as pltpu
from jax.experimental.pallas import tpu_sc as plsc

D_MODEL = 64
SCALE = 8.0

BATCH = 4096
SEQ = 200
HSEQ = SEQ // 2
NUM_CORES = 2
NUM_SUBCORES = 16
NW = NUM_CORES * NUM_SUBCORES
B_LOCAL = BATCH // NW
NBUF = 2
ROW_UNROLL = 8


def _build():
  mesh = plsc.VectorSubcoreMesh(core_axis_name="c", subcore_axis_name="s")

  @functools.partial(
      pl.kernel,
      mesh=mesh,
      out_type=jax.ShapeDtypeStruct((BATCH * SEQ, D_MODEL), jnp.float32),
      scratch_types=[
          pltpu.VMEM((SEQ, B_LOCAL), jnp.int32),
          pltpu.VMEM((NBUF, 2 * B_LOCAL, D_MODEL), jnp.float32),
          pltpu.VMEM((NBUF, 2 * B_LOCAL), jnp.int32),
          [pltpu.SemaphoreType.DMA] * NBUF,
          [pltpu.SemaphoreType.DMA] * NBUF,
      ],
      compiler_params=pltpu.CompilerParams(use_tc_tiling_on_sc=False),
  )
  def emb(xt_hbm, table_hbm, out_hbm, xt_v, rv, oidx_v, gsems, ssems):
    wid = lax.axis_index("s") * NUM_CORES + lax.axis_index("c")
    b0 = wid * B_LOCAL

    pltpu.sync_copy(xt_hbm.at[:, pl.ds(b0, B_LOCAL)], xt_v)

    lanes200 = lax.iota(jnp.int32, 16) * SEQ

    def start_gather(b, m):
      pltpu.make_async_copy(
          table_hbm.at[xt_v.at[2 * m]],
          rv.at[b, pl.ds(0, B_LOCAL)],
          gsems[b],
      ).start()
      pltpu.make_async_copy(
          table_hbm.at[xt_v.at[2 * m + 1]],
          rv.at[b, pl.ds(B_LOCAL, B_LOCAL)],
          gsems[b],
      ).start()
      for g in range(B_LOCAL // 16):
        base = (b0 + g * 16) * SEQ
        oidx_v[b, pl.ds(g * 16, 16)] = lanes200 + (base + 2 * m)
        oidx_v[b, pl.ds(2 * B_LOCAL // 2 + g * 16, 16)] = lanes200 + (base + 2 * m + 1)

    def wait_gather(b, m):
      pltpu.make_async_copy(
          table_hbm.at[xt_v.at[2 * m]],
          rv.at[b, pl.ds(0, B_LOCAL)],
          gsems[b],
      ).wait()
      pltpu.make_async_copy(
          table_hbm.at[xt_v.at[2 * m + 1]],
          rv.at[b, pl.ds(B_LOCAL, B_LOCAL)],
          gsems[b],
      ).wait()

    for b in range(NBUF):
      start_gather(b, b)

    def scale_buf(b):
      def body(i, c2):
        r0 = i * ROW_UNROLL
        for k in range(ROW_UNROLL):
          for j in range(D_MODEL // 16):
            s = pl.ds(j * 16, 16)
            rv[b, r0 + k, s] = rv[b, r0 + k, s] * SCALE
        return c2

      lax.fori_loop(0, 2 * B_LOCAL // ROW_UNROLL, body, 0)

    def outer(i, carry):
      for b in range(NBUF):
        m = i * NBUF + b
        wait_gather(b, m)
        scale_buf(b)
        pltpu.make_async_copy(
            rv.at[b], out_hbm.at[oidx_v.at[b]], ssems[b]
        ).start()

        @pl.when(m + NBUF < HSEQ)
        def _():
          pltpu.make_async_copy(
              rv.at[b], out_hbm.at[oidx_v.at[b]], ssems[b]
          ).wait()
          start_gather(b, m + NBUF)

      return carry

    lax.fori_loop(0, HSEQ // NBUF, outer, 0)

    for b in range(NBUF):
      pltpu.make_async_copy(
          rv.at[b], out_hbm.at[oidx_v.at[b]], ssems[b]
      ).wait()

  return emb


_emb = _build()


@jax.jit
def kernel(x, lut):
  out = _emb(x.T, lut)
  return out.reshape(BATCH, SEQ, D_MODEL)


# speedup vs baseline: 1.6438x; 1.0135x over previous
import functools

import jax
import jax.numpy as jnp
from jax import lax
from jax.experimental import pallas as pl
from jax.experimental.pallas import tpu as pltpu
from jax.experimental.pallas import tpu_sc as plsc

D_MODEL = 64
SCALE = 8.0

BATCH = 4096
SEQ = 200
HSEQ = SEQ // 2
NUM_CORES = 2
NUM_SUBCORES = 16
NW = NUM_CORES * NUM_SUBCORES
B_LOCAL = BATCH // NW
NBUF = 4
ROW_UNROLL = 8


def _build():
  mesh = plsc.VectorSubcoreMesh(core_axis_name="c", subcore_axis_name="s")

  @functools.partial(
      pl.kernel,
      mesh=mesh,
      out_type=jax.ShapeDtypeStruct((BATCH * SEQ, D_MODEL), jnp.float32),
      scratch_types=[
          pltpu.VMEM((SEQ, B_LOCAL), jnp.int32),
          pltpu.VMEM((NBUF, 2 * B_LOCAL, D_MODEL), jnp.float32),
          pltpu.VMEM((NBUF, 2 * B_LOCAL), jnp.int32),
          [pltpu.SemaphoreType.DMA] * NBUF,
          [pltpu.SemaphoreType.DMA] * NBUF,
      ],
      compiler_params=pltpu.CompilerParams(use_tc_tiling_on_sc=False),
  )
  def emb(xt_hbm, table_hbm, out_hbm, xt_v, rv, oidx_v, gsems, ssems):
    wid = lax.axis_index("s") * NUM_CORES + lax.axis_index("c")
    b0 = wid * B_LOCAL

    pltpu.sync_copy(xt_hbm.at[:, pl.ds(b0, B_LOCAL)], xt_v)

    lanes200 = lax.iota(jnp.int32, 16) * SEQ

    def start_gather(b, m):
      pltpu.make_async_copy(
          table_hbm.at[xt_v.at[2 * m]],
          rv.at[b, pl.ds(0, B_LOCAL)],
          gsems[b],
      ).start()
      pltpu.make_async_copy(
          table_hbm.at[xt_v.at[2 * m + 1]],
          rv.at[b, pl.ds(B_LOCAL, B_LOCAL)],
          gsems[b],
      ).start()
      for g in range(B_LOCAL // 16):
        base = (b0 + g * 16) * SEQ
        oidx_v[b, pl.ds(g * 16, 16)] = lanes200 + (base + 2 * m)
        oidx_v[b, pl.ds(2 * B_LOCAL // 2 + g * 16, 16)] = lanes200 + (base + 2 * m + 1)

    def wait_gather(b, m):
      pltpu.make_async_copy(
          table_hbm.at[xt_v.at[2 * m]],
          rv.at[b, pl.ds(0, B_LOCAL)],
          gsems[b],
      ).wait()
      pltpu.make_async_copy(
          table_hbm.at[xt_v.at[2 * m + 1]],
          rv.at[b, pl.ds(B_LOCAL, B_LOCAL)],
          gsems[b],
      ).wait()

    for b in range(NBUF):
      start_gather(b, b)

    def scale_buf(b):
      def body(i, c2):
        r0 = i * ROW_UNROLL
        for k in range(ROW_UNROLL):
          for j in range(D_MODEL // 16):
            s = pl.ds(j * 16, 16)
            rv[b, r0 + k, s] = rv[b, r0 + k, s] * SCALE
        return c2

      lax.fori_loop(0, 2 * B_LOCAL // ROW_UNROLL, body, 0)

    def outer(i, carry):
      for b in range(NBUF):
        m = i * NBUF + b
        wait_gather(b, m)
        scale_buf(b)
        pltpu.make_async_copy(
            rv.at[b], out_hbm.at[oidx_v.at[b]], ssems[b]
        ).start()

        @pl.when(m + NBUF < HSEQ)
        def _():
          pltpu.make_async_copy(
              rv.at[b], out_hbm.at[oidx_v.at[b]], ssems[b]
          ).wait()
          start_gather(b, m + NBUF)

      return carry

    lax.fori_loop(0, HSEQ // NBUF, outer, 0)

    for b in range(NBUF):
      pltpu.make_async_copy(
          rv.at[b], out_hbm.at[oidx_v.at[b]], ssems[b]
      ).wait()

  return emb


_emb = _build()


@jax.jit
def kernel(x, lut):
  out = _emb(x.T, lut)
  return out.reshape(BATCH, SEQ, D_MODEL)
